# Initial kernel scaffold; baseline (speedup 1.0000x reference)
import functools
import jax
import jax.numpy as jnp
from jax import lax
from jax.experimental import pallas as pl
from jax.experimental.pallas import tpu as pltpu
from jax.experimental.pallas import tpu_sc as plsc

F32 = jnp.float32
NN = 10000
NE = 320000
DD = 128
HH = 8
EP = NE + NN
NC = 2
NS = 16
NW = NC * NS
CB = 32
PW = -(-(-(-EP // NW)) // CB) * CB
EPAD = PW * NW
NCHUNK = PW // CB
RPT = NN // NS
RB = 1000
GRID = NN // RB

_mesh = plsc.VectorSubcoreMesh(core_axis_name="c", subcore_axis_name="s")


def _pre_body(x_ref, w1_ref, as1_ref, ad1_ref,
              h1_ref, sa_ref, da_ref, sm_ref, dm_ref):
    i = pl.program_id(0)
    h = jnp.dot(x_ref[...], w1_ref[...], preferred_element_type=F32)
    h1_ref[...] = h
    z = jnp.zeros((RB, 8), F32)
    sa_cols = [(h[:, 16 * k:16 * (k + 1)] * as1_ref[k, :][None, :]
                ).sum(-1, keepdims=True) for k in range(HH)]
    da_cols = [(h[:, 16 * k:16 * (k + 1)] * ad1_ref[k, :][None, :]
                ).sum(-1, keepdims=True) for k in range(HH)]
    sa = jnp.concatenate(sa_cols + [z], axis=1)
    da = jnp.concatenate(da_cols + [z], axis=1)
    sa_ref[...] = sa
    da_ref[...] = da
    bs = jnp.max(sa, axis=0, keepdims=True)
    bd = jnp.max(da, axis=0, keepdims=True)

    @pl.when(i == 0)
    def _():
        sm_ref[...] = bs
        dm_ref[...] = bd

    @pl.when(i > 0)
    def _():
        sm_ref[...] = jnp.maximum(sm_ref[...], bs)
        dm_ref[...] = jnp.maximum(dm_ref[...], bd)


_tc_pre = pl.pallas_call(
    _pre_body,
    grid=(GRID,),
    in_specs=[pl.BlockSpec((RB, DD), lambda i: (i, 0)),
              pl.BlockSpec((DD, HH * 16), lambda i: (0, 0)),
              pl.BlockSpec((HH, 16), lambda i: (0, 0)),
              pl.BlockSpec((HH, 16), lambda i: (0, 0))],
    out_specs=[pl.BlockSpec((RB, DD), lambda i: (i, 0)),
               pl.BlockSpec((RB, 16), lambda i: (i, 0)),
               pl.BlockSpec((RB, 16), lambda i: (i, 0)),
               pl.BlockSpec((1, 16), lambda i: (0, 0)),
               pl.BlockSpec((1, 16), lambda i: (0, 0))],
    out_shape=[jax.ShapeDtypeStruct((NN, DD), F32),
               jax.ShapeDtypeStruct((NN, 16), F32),
               jax.ShapeDtypeStruct((NN, 16), F32),
               jax.ShapeDtypeStruct((1, 16), F32),
               jax.ShapeDtypeStruct((1, 16), F32)],
)


def _mid_body(acc_ref, den_ref, b1_ref, w2_ref, as2_ref, ad2_ref,
              h2_ref, sa_ref, da_ref, sm_ref, dm_ref):
    i = pl.program_id(0)
    acc = acc_ref[0] + acc_ref[1]
    den = den_ref[0] + den_ref[1]
    cols = []
    for k in range(HH):
        inv = 1.0 / (den[:, k:k + 1] + 1e-16)
        cols.append(acc[:, 16 * k:16 * (k + 1)] * inv)
    x2 = jnp.maximum(jnp.concatenate(cols, axis=1) + b1_ref[...], 0.0)
    h2 = jnp.dot(x2, w2_ref[...], preferred_element_type=F32)
    h2_ref[...] = h2
    z = jnp.zeros((RB, 8), F32)
    sa_cols = [(h2[:, DD * k:DD * (k + 1)] * as2_ref[k, :][None, :]
                ).sum(-1, keepdims=True) for k in range(HH)]
    da_cols = [(h2[:, DD * k:DD * (k + 1)] * ad2_ref[k, :][None, :]
                ).sum(-1, keepdims=True) for k in range(HH)]
    sa = jnp.concatenate(sa_cols + [z], axis=1)
    da = jnp.concatenate(da_cols + [z], axis=1)
    sa_ref[...] = sa
    da_ref[...] = da
    bs = jnp.max(sa, axis=0, keepdims=True)
    bd = jnp.max(da, axis=0, keepdims=True)

    @pl.when(i == 0)
    def _():
        sm_ref[...] = bs
        dm_ref[...] = bd

    @pl.when(i > 0)
    def _():
        sm_ref[...] = jnp.maximum(sm_ref[...], bs)
        dm_ref[...] = jnp.maximum(dm_ref[...], bd)


_tc_mid = pl.pallas_call(
    _mid_body,
    grid=(GRID,),
    in_specs=[pl.BlockSpec((NC, RB, DD), lambda i: (0, i, 0)),
              pl.BlockSpec((NC, RB, 16), lambda i: (0, i, 0)),
              pl.BlockSpec((1, DD), lambda i: (0, 0)),
              pl.BlockSpec((DD, HH * DD), lambda i: (0, 0)),
              pl.BlockSpec((HH, DD), lambda i: (0, 0)),
              pl.BlockSpec((HH, DD), lambda i: (0, 0))],
    out_specs=[pl.BlockSpec((RB, HH * DD), lambda i: (i, 0)),
               pl.BlockSpec((RB, 16), lambda i: (i, 0)),
               pl.BlockSpec((RB, 16), lambda i: (i, 0)),
               pl.BlockSpec((1, 16), lambda i: (0, 0)),
               pl.BlockSpec((1, 16), lambda i: (0, 0))],
    out_shape=[jax.ShapeDtypeStruct((NN, HH * DD), F32),
               jax.ShapeDtypeStruct((NN, 16), F32),
               jax.ShapeDtypeStruct((NN, 16), F32),
               jax.ShapeDtypeStruct((1, 16), F32),
               jax.ShapeDtypeStruct((1, 16), F32)],
)


def _recip_body(den_ref, inv_ref):
    inv_ref[...] = 1.0 / (den_ref[0] + den_ref[1] + 1e-16)


_tc_recip = pl.pallas_call(
    _recip_body,
    out_shape=jax.ShapeDtypeStruct((NN, 16), F32),
)


def _post_body(acc_ref, b2_ref, o_ref):
    o_ref[...] = (acc_ref[0] + acc_ref[1]) * (1.0 / HH) + b2_ref[...]


_tc_post = pl.pallas_call(
    _post_body,
    grid=(GRID,),
    in_specs=[pl.BlockSpec((NC, RB, DD), lambda i: (0, i, 0)),
              pl.BlockSpec((1, DD), lambda i: (0, 0))],
    out_specs=pl.BlockSpec((RB, DD), lambda i: (i, 0)),
    out_shape=jax.ShapeDtypeStruct((NN, DD), F32),
)


@functools.partial(
    pl.kernel, mesh=_mesh,
    out_type=[jax.ShapeDtypeStruct((NC, NN, DD), F32),
              jax.ShapeDtypeStruct((NC, NN, 16), F32)],
    scratch_types=[
        pltpu.VMEM((CB,), jnp.int32),
        pltpu.VMEM((CB,), jnp.int32),
        pltpu.VMEM((CB, 16), F32),
        pltpu.VMEM((CB, 16), F32),
        pltpu.VMEM((CB, 16), F32),
        pltpu.VMEM((CB, DD), F32),
        pltpu.VMEM((CB, DD), F32),
        pltpu.VMEM((1, 16), F32),
        pltpu.VMEM((1, 16), F32),
        pltpu.VMEM_SHARED((NN, DD), F32),
        pltpu.VMEM_SHARED((NN, 16), F32),
        pltpu.SemaphoreType.DMA,
    ],
)
def _sc_layer1(src_hbm, dst_hbm, h1_hbm, sa_hbm, da_hbm, sm_hbm, dm_hbm,
               zacc_hbm, zden_hbm,
               acc_out, den_out,
               src_v, dst_v, sa_v, da_v, al_v, hr_v, or_v, smv, dmv,
               acc_sh, den_sh, sem):
    c = lax.axis_index("c")
    s = lax.axis_index("s")
    wid = s * NC + c
    pltpu.sync_copy(zacc_hbm.at[pl.ds(s * RPT, RPT)],
                    acc_sh.at[pl.ds(s * RPT, RPT)])
    pltpu.sync_copy(zden_hbm.at[pl.ds(s * RPT, RPT)],
                    den_sh.at[pl.ds(s * RPT, RPT)])
    pltpu.sync_copy(sm_hbm, smv)
    pltpu.sync_copy(dm_hbm, dmv)
    plsc.subcore_barrier()
    mv = smv[0, :] + dmv[0, :]
    headmask = lax.iota(jnp.int32, 16) < HH
    base0 = wid * PW

    def chunk(ci, carry):
        base = base0 + ci * CB
        pltpu.sync_copy(src_hbm.at[pl.ds(base, CB)], src_v)
        pltpu.sync_copy(dst_hbm.at[pl.ds(base, CB)], dst_v)
        cp1 = pltpu.async_copy(sa_hbm.at[src_v], sa_v, sem)
        cp2 = pltpu.async_copy(da_hbm.at[dst_v], da_v, sem)
        cp3 = pltpu.async_copy(h1_hbm.at[src_v], hr_v, sem)
        cp1.wait()
        cp2.wait()
        cp3.wait()

        def edge(b, carry2):
            a = sa_v[b, :] + da_v[b, :]
            a = jnp.maximum(a, 0.2 * a)
            a = jnp.exp(a - mv)
            valid = (base + b) < EP
            a = jnp.where(jnp.logical_and(headmask, valid), a, 0.0)
            al_v[b, :] = a
            for k in range(HH):
                w = al_v[b, k]
                or_v[b, pl.ds(16 * k, 16)] = w * hr_v[b, pl.ds(16 * k, 16)]
            return carry2

        lax.fori_loop(0, CB, edge, 0)
        pltpu.sync_copy(al_v, den_sh.at[dst_v], add=True)
        pltpu.sync_copy(or_v, acc_sh.at[dst_v], add=True)
        return carry

    lax.fori_loop(0, NCHUNK, chunk, 0)
    plsc.subcore_barrier()
    pltpu.sync_copy(acc_sh.at[pl.ds(s * RPT, RPT)],
                    acc_out.at[c, pl.ds(s * RPT, RPT)])
    pltpu.sync_copy(den_sh.at[pl.ds(s * RPT, RPT)],
                    den_out.at[c, pl.ds(s * RPT, RPT)])


@functools.partial(
    pl.kernel, mesh=_mesh,
    out_type=[jax.ShapeDtypeStruct((NC, NN, 16), F32),
              jax.ShapeDtypeStruct((EPAD, 16), F32)],
    scratch_types=[
        pltpu.VMEM((CB,), jnp.int32),
        pltpu.VMEM((CB,), jnp.int32),
        pltpu.VMEM((CB, 16), F32),
        pltpu.VMEM((CB, 16), F32),
        pltpu.VMEM((CB, 16), F32),
        pltpu.VMEM((1, 16), F32),
        pltpu.VMEM((1, 16), F32),
        pltpu.VMEM_SHARED((NN, 16), F32),
        pltpu.SemaphoreType.DMA,
    ],
)
def _sc_alpha2(src_hbm, dst_hbm, sa_hbm, da_hbm, sm_hbm, dm_hbm, zden_hbm,
               den_out, ab_out,
               src_v, dst_v, sa_v, da_v, al_v, smv, dmv, den_sh, sem):
    c = lax.axis_index("c")
    s = lax.axis_index("s")
    wid = s * NC + c
    pltpu.sync_copy(zden_hbm.at[pl.ds(s * RPT, RPT)],
                    den_sh.at[pl.ds(s * RPT, RPT)])
    pltpu.sync_copy(sm_hbm, smv)
    pltpu.sync_copy(dm_hbm, dmv)
    plsc.subcore_barrier()
    mv = smv[0, :] + dmv[0, :]
    headmask = lax.iota(jnp.int32, 16) < HH
    base0 = wid * PW

    def chunk(ci, carry):
        base = base0 + ci * CB
        pltpu.sync_copy(src_hbm.at[pl.ds(base, CB)], src_v)
        pltpu.sync_copy(dst_hbm.at[pl.ds(base, CB)], dst_v)
        cp1 = pltpu.async_copy(sa_hbm.at[src_v], sa_v, sem)
        cp2 = pltpu.async_copy(da_hbm.at[dst_v], da_v, sem)
        cp1.wait()
        cp2.wait()

        def edge(b, carry2):
            a = sa_v[b, :] + da_v[b, :]
            a = jnp.maximum(a, 0.2 * a)
            a = jnp.exp(a - mv)
            valid = (base + b) < EP
            a = jnp.where(jnp.logical_and(headmask, valid), a, 0.0)
            al_v[b, :] = a
            return carry2

        lax.fori_loop(0, CB, edge, 0)
        pltpu.sync_copy(al_v, den_sh.at[dst_v], add=True)
        pltpu.sync_copy(al_v, ab_out.at[pl.ds(base, CB)])
        return carry

    lax.fori_loop(0, NCHUNK, chunk, 0)
    plsc.subcore_barrier()
    pltpu.sync_copy(den_sh.at[pl.ds(s * RPT, RPT)],
                    den_out.at[c, pl.ds(s * RPT, RPT)])


@functools.partial(
    pl.kernel, mesh=_mesh,
    out_type=[jax.ShapeDtypeStruct((NC, NN, DD), F32)],
    scratch_types=[
        pltpu.VMEM((CB,), jnp.int32),
        pltpu.VMEM((CB,), jnp.int32),
        pltpu.VMEM((CB, 16), F32),
        pltpu.VMEM((CB, 16), F32),
        pltpu.VMEM((CB, 16), F32),
        pltpu.VMEM((CB, HH * DD), F32),
        pltpu.VMEM((CB, DD), F32),
        pltpu.VMEM_SHARED((NN, DD), F32),
        pltpu.SemaphoreType.DMA,
    ],
)
def _sc_agg2(src_hbm, dst_hbm, h2_hbm, ab_hbm, inv_hbm, zacc_hbm,
             acc_out,
             src_v, dst_v, al_v, iv_v, w_v, hr_v, or_v, acc_sh, sem):
    c = lax.axis_index("c")
    s = lax.axis_index("s")
    wid = s * NC + c
    pltpu.sync_copy(zacc_hbm.at[pl.ds(s * RPT, RPT)],
                    acc_sh.at[pl.ds(s * RPT, RPT)])
    plsc.subcore_barrier()
    base0 = wid * PW

    def chunk(ci, carry):
        base = base0 + ci * CB
        pltpu.sync_copy(src_hbm.at[pl.ds(base, CB)], src_v)
        pltpu.sync_copy(dst_hbm.at[pl.ds(base, CB)], dst_v)
        cp1 = pltpu.async_copy(h2_hbm.at[src_v], hr_v, sem)
        cp2 = pltpu.async_copy(inv_hbm.at[dst_v], iv_v, sem)
        pltpu.sync_copy(ab_hbm.at[pl.ds(base, CB)], al_v)
        cp1.wait()
        cp2.wait()

        def edge(b, carry2):
            w_v[b, :] = al_v[b, :] * iv_v[b, :]
            for j in range(HH):
                acc = w_v[b, 0] * hr_v[b, pl.ds(16 * j, 16)]
                for k in range(1, HH):
                    acc = acc + w_v[b, k] * hr_v[b, pl.ds(DD * k + 16 * j, 16)]
                or_v[b, pl.ds(16 * j, 16)] = acc
            return carry2

        lax.fori_loop(0, CB, edge, 0)
        pltpu.sync_copy(or_v, acc_sh.at[dst_v], add=True)
        return carry

    lax.fori_loop(0, NCHUNK, chunk, 0)
    plsc.subcore_barrier()
    pltpu.sync_copy(acc_sh.at[pl.ds(s * RPT, RPT)],
                    acc_out.at[c, pl.ds(s * RPT, RPT)])


def kernel(x, edge_index, W1, as1, ad1, b1, W2, as2, ad2, b2):
    loop = jnp.arange(NN, dtype=jnp.int32)
    pad = jnp.zeros((EPAD - EP,), jnp.int32)
    src = jnp.concatenate([edge_index[0].astype(jnp.int32), loop, pad])
    dst = jnp.concatenate([edge_index[1].astype(jnp.int32), loop, pad])
    zacc = jnp.zeros((NN, DD), F32)
    zden = jnp.zeros((NN, 16), F32)
    h1, sa1, da1, sm1, dm1 = _tc_pre(x, W1, as1, ad1)
    acc1, den1 = _sc_layer1(src, dst, h1, sa1, da1, sm1, dm1, zacc, zden)
    h2, sa2, da2, sm2, dm2 = _tc_mid(acc1, den1, b1.reshape(1, DD),
                                     W2, as2, ad2)
    den2, ab = _sc_alpha2(src, dst, sa2, da2, sm2, dm2, zden)
    inv2 = _tc_recip(den2)
    (acc2,) = _sc_agg2(src, dst, h2, ab, inv2, zacc)
    return _tc_post(acc2, b2.reshape(1, DD))


# speedup vs baseline: 17.1396x; 17.1396x over previous
import functools
import jax
import jax.numpy as jnp
from jax import lax
from jax.experimental import pallas as pl
from jax.experimental.pallas import tpu as pltpu
from jax.experimental.pallas import tpu_sc as plsc

F32 = jnp.float32
NN = 10000
NE = 320000
DD = 128
HH = 8
EP = NE + NN
NC = 2
NS = 16
NW = NC * NS
CB = 32
PW = ((EP + NW - 1) // NW + CB - 1) // CB * CB
EPAD = PW * NW
NCHUNK = PW // CB
NP = 10240
RPT = NP // NS
RB = 1000
GRID = NN // RB

_mesh = plsc.VectorSubcoreMesh(core_axis_name="c", subcore_axis_name="s")


def _pre_body(x_ref, w1_ref, as1_ref, ad1_ref,
              h1_ref, sa_ref, da_ref, sm_ref, dm_ref):
    i = pl.program_id(0)
    h = jnp.dot(x_ref[...], w1_ref[...], preferred_element_type=F32)
    h1_ref[...] = h
    z = jnp.zeros((RB, 120), F32)
    sa_cols = [(h[:, 16 * k:16 * (k + 1)] * as1_ref[k, :][None, :]
                ).sum(-1, keepdims=True) for k in range(HH)]
    da_cols = [(h[:, 16 * k:16 * (k + 1)] * ad1_ref[k, :][None, :]
                ).sum(-1, keepdims=True) for k in range(HH)]
    sa = jnp.concatenate(sa_cols + [z], axis=1)
    da = jnp.concatenate(da_cols + [z], axis=1)
    sa_ref[...] = sa
    da_ref[...] = da
    bs = jnp.max(sa, axis=0, keepdims=True)[:, :16]
    bd = jnp.max(da, axis=0, keepdims=True)[:, :16]

    @pl.when(i == 0)
    def _():
        sm_ref[...] = bs
        dm_ref[...] = bd

    @pl.when(i > 0)
    def _():
        sm_ref[...] = jnp.maximum(sm_ref[...], bs)
        dm_ref[...] = jnp.maximum(dm_ref[...], bd)


_tc_pre = pl.pallas_call(
    _pre_body,
    grid=(GRID,),
    in_specs=[pl.BlockSpec((RB, DD), lambda i: (i, 0)),
              pl.BlockSpec((DD, HH * 16), lambda i: (0, 0)),
              pl.BlockSpec((HH, 16), lambda i: (0, 0)),
              pl.BlockSpec((HH, 16), lambda i: (0, 0))],
    out_specs=[pl.BlockSpec((RB, DD), lambda i: (i, 0)),
               pl.BlockSpec((RB, DD), lambda i: (i, 0)),
               pl.BlockSpec((RB, DD), lambda i: (i, 0)),
               pl.BlockSpec((1, 16), lambda i: (0, 0)),
               pl.BlockSpec((1, 16), lambda i: (0, 0))],
    out_shape=[jax.ShapeDtypeStruct((NN, DD), F32),
               jax.ShapeDtypeStruct((NN, DD), F32),
               jax.ShapeDtypeStruct((NN, DD), F32),
               jax.ShapeDtypeStruct((1, 16), F32),
               jax.ShapeDtypeStruct((1, 16), F32)],
)


def _mid_body(acc_ref, den_ref, b1_ref, w2_ref, as2_ref, ad2_ref,
              h2_ref, sa_ref, da_ref, sm_ref, dm_ref):
    i = pl.program_id(0)
    acc = acc_ref[0] + acc_ref[1]
    den = den_ref[0] + den_ref[1]
    cols = []
    for k in range(HH):
        inv = 1.0 / (den[:, k:k + 1] + 1e-16)
        cols.append(acc[:, 16 * k:16 * (k + 1)] * inv)
    x2 = jnp.maximum(jnp.concatenate(cols, axis=1) + b1_ref[...], 0.0)
    h2 = jnp.dot(x2, w2_ref[...], preferred_element_type=F32)
    h2_ref[...] = h2
    z = jnp.zeros((RB, 120), F32)
    sa_cols = [(h2[:, DD * k:DD * (k + 1)] * as2_ref[k, :][None, :]
                ).sum(-1, keepdims=True) for k in range(HH)]
    da_cols = [(h2[:, DD * k:DD * (k + 1)] * ad2_ref[k, :][None, :]
                ).sum(-1, keepdims=True) for k in range(HH)]
    sa = jnp.concatenate(sa_cols + [z], axis=1)
    da = jnp.concatenate(da_cols + [z], axis=1)
    sa_ref[...] = sa
    da_ref[...] = da
    bs = jnp.max(sa, axis=0, keepdims=True)[:, :16]
    bd = jnp.max(da, axis=0, keepdims=True)[:, :16]

    @pl.when(i == 0)
    def _():
        sm_ref[...] = bs
        dm_ref[...] = bd

    @pl.when(i > 0)
    def _():
        sm_ref[...] = jnp.maximum(sm_ref[...], bs)
        dm_ref[...] = jnp.maximum(dm_ref[...], bd)


_tc_mid = pl.pallas_call(
    _mid_body,
    grid=(GRID,),
    in_specs=[pl.BlockSpec((NC, RB, DD), lambda i: (0, i, 0)),
              pl.BlockSpec((NC, RB, DD), lambda i: (0, i, 0)),
              pl.BlockSpec((1, DD), lambda i: (0, 0)),
              pl.BlockSpec((DD, HH * DD), lambda i: (0, 0)),
              pl.BlockSpec((HH, DD), lambda i: (0, 0)),
              pl.BlockSpec((HH, DD), lambda i: (0, 0))],
    out_specs=[pl.BlockSpec((RB, HH * DD), lambda i: (i, 0)),
               pl.BlockSpec((RB, DD), lambda i: (i, 0)),
               pl.BlockSpec((RB, DD), lambda i: (i, 0)),
               pl.BlockSpec((1, 16), lambda i: (0, 0)),
               pl.BlockSpec((1, 16), lambda i: (0, 0))],
    out_shape=[jax.ShapeDtypeStruct((NN, HH * DD), F32),
               jax.ShapeDtypeStruct((NN, DD), F32),
               jax.ShapeDtypeStruct((NN, DD), F32),
               jax.ShapeDtypeStruct((1, 16), F32),
               jax.ShapeDtypeStruct((1, 16), F32)],
)


def _recip_body(den_ref, inv_ref):
    inv16 = 1.0 / (den_ref[0, :, :16] + den_ref[1, :, :16] + 1e-16)
    inv_ref[...] = jnp.concatenate([inv16, jnp.zeros((NP, 112), F32)], axis=1)


_tc_recip = pl.pallas_call(
    _recip_body,
    out_shape=jax.ShapeDtypeStruct((NP, DD), F32),
)


def _post_body(acc_ref, b2_ref, o_ref):
    o_ref[...] = (acc_ref[0] + acc_ref[1]) * (1.0 / HH) + b2_ref[...]


_tc_post = pl.pallas_call(
    _post_body,
    grid=(GRID,),
    in_specs=[pl.BlockSpec((NC, RB, DD), lambda i: (0, i, 0)),
              pl.BlockSpec((1, DD), lambda i: (0, 0))],
    out_specs=pl.BlockSpec((RB, DD), lambda i: (i, 0)),
    out_shape=jax.ShapeDtypeStruct((NN, DD), F32),
)


@functools.partial(
    pl.kernel, mesh=_mesh,
    out_type=[jax.ShapeDtypeStruct((NC, NP, DD), F32),
              jax.ShapeDtypeStruct((EPAD, 16), F32)],
    scratch_types=[
        pltpu.VMEM((CB,), jnp.int32),
        pltpu.VMEM((CB,), jnp.int32),
        pltpu.VMEM((CB,), jnp.int32),
        pltpu.VMEM((CB, DD), F32),
        pltpu.VMEM((CB, DD), F32),
        pltpu.VMEM((CB, DD), F32),
        pltpu.VMEM((CB, 16), F32),
        pltpu.VMEM((1, 16), F32),
        pltpu.VMEM((1, 16), F32),
        pltpu.VMEM_SHARED((NP, DD), F32),
        pltpu.SemaphoreType.DMA,
    ],
)
def _sc_alpha(src_hbm, dst_hbm, dsts_hbm, sa_hbm, da_hbm, sm_hbm, dm_hbm,
              zacc_hbm,
              den_out, ab_out,
              src_v, dst_v, dsts_v, sa_v, da_v, al_v, al16_v, smv, dmv,
              den_sh, sem):
    c = lax.axis_index("c")
    s = lax.axis_index("s")
    wid = s * NC + c
    pltpu.sync_copy(zacc_hbm.at[pl.ds(s * RPT, RPT)],
                    den_sh.at[pl.ds(s * RPT, RPT)])
    pltpu.sync_copy(zacc_hbm.at[pl.ds(0, CB)], al_v)
    pltpu.sync_copy(sm_hbm, smv)
    pltpu.sync_copy(dm_hbm, dmv)
    plsc.subcore_barrier()
    mv = smv[0, :] + dmv[0, :]
    base0 = wid * PW

    def chunk(ci, carry):
        base = base0 + ci * CB
        pltpu.sync_copy(src_hbm.at[pl.ds(base, CB)], src_v)
        pltpu.sync_copy(dst_hbm.at[pl.ds(base, CB)], dst_v)
        pltpu.sync_copy(dsts_hbm.at[pl.ds(base, CB)], dsts_v)
        cp1 = pltpu.async_copy(sa_hbm.at[src_v], sa_v, sem)
        cp2 = pltpu.async_copy(da_hbm.at[dst_v], da_v, sem)
        cp1.wait()
        cp2.wait()

        def edge(b, carry2):
            a = sa_v[b, pl.ds(0, 16)] + da_v[b, pl.ds(0, 16)]
            a = jnp.maximum(a, 0.2 * a)
            a = jnp.exp(a - mv)
            al_v[b, pl.ds(0, 16)] = a
            al16_v[b, :] = a
            return carry2

        lax.fori_loop(0, CB, edge, 0)
        pltpu.sync_copy(al_v, den_sh.at[dsts_v], add=True)
        pltpu.sync_copy(al16_v, ab_out.at[pl.ds(base, CB)])
        return carry

    lax.fori_loop(0, NCHUNK, chunk, 0)
    plsc.subcore_barrier()
    pltpu.sync_copy(den_sh.at[pl.ds(s * RPT, RPT)],
                    den_out.at[c, pl.ds(s * RPT, RPT)])


@functools.partial(
    pl.kernel, mesh=_mesh,
    out_type=[jax.ShapeDtypeStruct((NC, NP, DD), F32)],
    scratch_types=[
        pltpu.VMEM((CB,), jnp.int32),
        pltpu.VMEM((CB,), jnp.int32),
        pltpu.VMEM((CB, 16), F32),
        pltpu.VMEM((CB, DD), F32),
        pltpu.VMEM((CB, DD), F32),
        pltpu.VMEM_SHARED((NP, DD), F32),
        pltpu.SemaphoreType.DMA,
    ],
)
def _sc_agg1(src_hbm, dsts_hbm, h1_hbm, ab_hbm, zacc_hbm,
             acc_out,
             src_v, dsts_v, al_v, hr_v, or_v, acc_sh, sem):
    c = lax.axis_index("c")
    s = lax.axis_index("s")
    wid = s * NC + c
    pltpu.sync_copy(zacc_hbm.at[pl.ds(s * RPT, RPT)],
                    acc_sh.at[pl.ds(s * RPT, RPT)])
    plsc.subcore_barrier()
    base0 = wid * PW

    def chunk(ci, carry):
        base = base0 + ci * CB
        pltpu.sync_copy(src_hbm.at[pl.ds(base, CB)], src_v)
        pltpu.sync_copy(dsts_hbm.at[pl.ds(base, CB)], dsts_v)
        cp1 = pltpu.async_copy(h1_hbm.at[src_v], hr_v, sem)
        pltpu.sync_copy(ab_hbm.at[pl.ds(base, CB)], al_v)
        cp1.wait()

        def edge(b, carry2):
            a = al_v[b, :]
            for k in range(HH):
                or_v[b, pl.ds(16 * k, 16)] = a[k] * hr_v[b, pl.ds(16 * k, 16)]
            return carry2

        lax.fori_loop(0, CB, edge, 0)
        pltpu.sync_copy(or_v, acc_sh.at[dsts_v], add=True)
        return carry

    lax.fori_loop(0, NCHUNK, chunk, 0)
    plsc.subcore_barrier()
    pltpu.sync_copy(acc_sh.at[pl.ds(s * RPT, RPT)],
                    acc_out.at[c, pl.ds(s * RPT, RPT)])


@functools.partial(
    pl.kernel, mesh=_mesh,
    out_type=[jax.ShapeDtypeStruct((NC, NP, DD), F32)],
    scratch_types=[
        pltpu.VMEM((CB,), jnp.int32),
        pltpu.VMEM((CB,), jnp.int32),
        pltpu.VMEM((CB,), jnp.int32),
        pltpu.VMEM((CB, 16), F32),
        pltpu.VMEM((CB, DD), F32),
        pltpu.VMEM((CB, HH * DD), F32),
        pltpu.VMEM((CB, DD), F32),
        pltpu.VMEM_SHARED((NP, DD), F32),
        pltpu.SemaphoreType.DMA,
    ],
)
def _sc_agg2(src_hbm, dst_hbm, dsts_hbm, h2_hbm, ab_hbm, inv_hbm, zacc_hbm,
             acc_out,
             src_v, dst_v, dsts_v, al_v, iv_v, hr_v, or_v, acc_sh, sem):
    c = lax.axis_index("c")
    s = lax.axis_index("s")
    wid = s * NC + c
    pltpu.sync_copy(zacc_hbm.at[pl.ds(s * RPT, RPT)],
                    acc_sh.at[pl.ds(s * RPT, RPT)])
    plsc.subcore_barrier()
    base0 = wid * PW

    def chunk(ci, carry):
        base = base0 + ci * CB
        pltpu.sync_copy(src_hbm.at[pl.ds(base, CB)], src_v)
        pltpu.sync_copy(dst_hbm.at[pl.ds(base, CB)], dst_v)
        pltpu.sync_copy(dsts_hbm.at[pl.ds(base, CB)], dsts_v)
        cp1 = pltpu.async_copy(h2_hbm.at[src_v], hr_v, sem)
        cp2 = pltpu.async_copy(inv_hbm.at[dst_v], iv_v, sem)
        pltpu.sync_copy(ab_hbm.at[pl.ds(base, CB)], al_v)
        cp1.wait()
        cp2.wait()

        def edge(b, carry2):
            w = al_v[b, :] * iv_v[b, pl.ds(0, 16)]
            for j in range(HH):
                acc = w[0] * hr_v[b, pl.ds(16 * j, 16)]
                for k in range(1, HH):
                    acc = acc + w[k] * hr_v[b, pl.ds(DD * k + 16 * j, 16)]
                or_v[b, pl.ds(16 * j, 16)] = acc
            return carry2

        lax.fori_loop(0, CB, edge, 0)
        pltpu.sync_copy(or_v, acc_sh.at[dsts_v], add=True)
        return carry

    lax.fori_loop(0, NCHUNK, chunk, 0)
    plsc.subcore_barrier()
    pltpu.sync_copy(acc_sh.at[pl.ds(s * RPT, RPT)],
                    acc_out.at[c, pl.ds(s * RPT, RPT)])


def kernel(x, edge_index, W1, as1, ad1, b1, W2, as2, ad2, b2):
    loop = jnp.arange(NN, dtype=jnp.int32)
    padg = jnp.zeros((EPAD - EP,), jnp.int32)
    pads = jnp.full((EPAD - EP,), NN, jnp.int32)
    src = jnp.concatenate([edge_index[0].astype(jnp.int32), loop, padg])
    dst = jnp.concatenate([edge_index[1].astype(jnp.int32), loop, padg])
    dsts = jnp.concatenate([edge_index[1].astype(jnp.int32), loop, pads])
    zacc = jnp.zeros((NP, DD), F32)
    h1, sa1, da1, sm1, dm1 = _tc_pre(x, W1, as1, ad1)
    den1, ab1 = _sc_alpha(src, dst, dsts, sa1, da1, sm1, dm1, zacc)
    (acc1,) = _sc_agg1(src, dsts, h1, ab1, zacc)
    h2, sa2, da2, sm2, dm2 = _tc_mid(acc1, den1, b1.reshape(1, DD),
                                     W2, as2, ad2)
    den2, ab2 = _sc_alpha(src, dst, dsts, sa2, da2, sm2, dm2, zacc)
    inv2 = _tc_recip(den2)
    (acc2,) = _sc_agg2(src, dst, dsts, h2, ab2, inv2, zacc)
    return _tc_post(acc2, b2.reshape(1, DD))
